# baseline (speedup 1.0000x reference)
import functools

import jax
import jax.numpy as jnp
from jax import lax
from jax.experimental import pallas as pl
from jax.experimental.pallas import tpu as pltpu
from jax.experimental.pallas import tpu_sc as plsc

MAXLEN = 200
VOCAB = 100000
EMBED = 64
BATCH = 4096

CHUNKS = [(0, 24), (24, 88), (112, 64), (176, 24)]
NSPLIT = len(CHUNKS)

NC, NS = 2, 16
NW = NC * NS
SEQ_PER_W = BATCH // NW
ADD_UNROLL = 4

_mesh = plsc.VectorSubcoreMesh(core_axis_name="c", subcore_axis_name="s")


def _make_sc_embed_chunk(l0, lchunk):
  lpairs = lchunk // 2
  @functools.partial(
      pl.kernel,
      out_type=jax.ShapeDtypeStruct((lpairs, BATCH, 128), jnp.float32),
      mesh=_mesh,
      compiler_params=pltpu.CompilerParams(use_tc_tiling_on_sc=False),
      scratch_types=[
          pltpu.VMEM((SEQ_PER_W, lchunk), jnp.int32),
          pltpu.VMEM((lchunk, EMBED), jnp.float32),
          pltpu.VMEM((lchunk, EMBED), jnp.float32),
          pltpu.VMEM((lchunk, EMBED), jnp.float32),
          pltpu.VMEM((lpairs, 128), jnp.float32),
          pltpu.VMEM((lpairs, 128), jnp.float32),
          pltpu.SemaphoreType.DMA,
          pltpu.SemaphoreType.DMA,
          pltpu.SemaphoreType.DMA,
          pltpu.SemaphoreType.DMA,
      ],
  )
  def _sc_embed_chunk(idx_hbm, tok_hbm, pos_hbm, out_hbm,
                      idx_all, pos_v, rows0, rows1, stg0, stg1,
                      gsem0, gsem1, wsem0, wsem1):
    wid = lax.axis_index("s") * NC + lax.axis_index("c")
    seq_base = wid * SEQ_PER_W
    rows = (rows0, rows1)
    stg = (stg0, stg1)
    gsem = (gsem0, gsem1)
    wsem = (wsem0, wsem1)

    pltpu.sync_copy(
        idx_hbm.at[pl.ds(seq_base, SEQ_PER_W), pl.ds(l0, lchunk)], idx_all)
    pltpu.sync_copy(pos_hbm.at[pl.ds(l0, lchunk)], pos_v)

    def start_gather(s, p):
        pltpu.async_copy(tok_hbm.at[idx_all.at[s]], rows[p], gsem[p])

    def drain_gather(p):
        pltpu.make_async_copy(tok_hbm.at[pl.ds(0, lchunk)], rows[p],
                              gsem[p]).wait()

    def add_and_write(s, p):
        @plsc.parallel_loop(0, lpairs, unroll=ADD_UNROLL)
        def _add(r2):
            for h in range(2):
                for c4 in range(EMBED // 16):
                    src = pl.ds(c4 * 16, 16)
                    dst = pl.ds(h * EMBED + c4 * 16, 16)
                    stg[p][r2, dst] = (rows[p][r2 * 2 + h, src]
                                       + pos_v[r2 * 2 + h, src])

        pltpu.async_copy(stg[p], out_hbm.at[pl.ds(0, lpairs), seq_base + s],
                         wsem[p])

    def drain_write(p):
        pltpu.make_async_copy(stg[p], out_hbm.at[pl.ds(0, lpairs), 0],
                              wsem[p]).wait()

    start_gather(0, 0)

    @pl.loop(0, SEQ_PER_W // 2)
    def _pipeline(h2):
        for par in (0, 1):
            s = h2 * 2 + par
            q = 1 - par

            @pl.when(s + 1 < SEQ_PER_W)
            def _():
                start_gather(s + 1, q)
            drain_gather(par)

            @pl.when(s >= 2)
            def _():
                drain_write(par)
            add_and_write(s, par)

    drain_write(0)
    drain_write(1)

  return _sc_embed_chunk


_sc_chunk_kernels = [_make_sc_embed_chunk(l0, lc) for l0, lc in CHUNKS]


def _tc_relayout_body(in_ref, _prev_ref, out_ref):
    t = in_ref[0]
    tt = jnp.transpose(t)
    out_ref[...] = tt.reshape(2, EMBED, BATCH)


def _tc_relayout_chunk(dense, out_buf, pair_off):
    lpairs = dense.shape[0]
    return pl.pallas_call(
        _tc_relayout_body,
        grid=(lpairs,),
        in_specs=[
            pl.BlockSpec((1, BATCH, 128), lambda i: (i, 0, 0)),
            pl.BlockSpec(memory_space=pl.ANY),
        ],
        out_specs=pl.BlockSpec((2, EMBED, BATCH),
                               lambda i, o=pair_off: (o + i, 0, 0)),
        out_shape=jax.ShapeDtypeStruct((MAXLEN, EMBED, BATCH), jnp.float32),
        input_output_aliases={1: 0},
    )(dense, out_buf)


def _tc_relayout_first_body(in_ref, out_ref):
    t = in_ref[0]
    tt = jnp.transpose(t)
    out_ref[...] = tt.reshape(2, EMBED, BATCH)


def _tc_relayout_first(dense):
    return pl.pallas_call(
        _tc_relayout_first_body,
        grid=(dense.shape[0],),
        in_specs=[pl.BlockSpec((1, BATCH, 128), lambda i: (i, 0, 0))],
        out_specs=pl.BlockSpec((2, EMBED, BATCH), lambda i: (i, 0, 0)),
        out_shape=jax.ShapeDtypeStruct((MAXLEN, EMBED, BATCH), jnp.float32),
    )(dense)


def kernel(x, token_table, pos_table):
    xi = x.astype(jnp.int32)
    chunks = [sc(xi, token_table, pos_table) for sc in _sc_chunk_kernels]
    out_t = _tc_relayout_first(chunks[0])
    pair_off = CHUNKS[0][1] // 2
    for k in range(1, NSPLIT):
        out_t = _tc_relayout_chunk(chunks[k], out_t, pair_off)
        pair_off += CHUNKS[k][1] // 2
    return out_t.transpose(2, 0, 1)


# speedup vs baseline: 1.0633x; 1.0633x over previous
import functools

import jax
import jax.numpy as jnp
from jax import lax
from jax.experimental import pallas as pl
from jax.experimental.pallas import tpu as pltpu
from jax.experimental.pallas import tpu_sc as plsc

MAXLEN = 200
VOCAB = 100000
EMBED = 64
BATCH = 4096

CHUNKS = [(0, 72), (72, 72), (144, 56)]
NSPLIT = len(CHUNKS)

NC, NS = 2, 16
NW = NC * NS
SEQ_PER_W = BATCH // NW
ADD_UNROLL = 4

_mesh = plsc.VectorSubcoreMesh(core_axis_name="c", subcore_axis_name="s")


def _make_sc_embed_chunk(l0, lchunk):
  lpairs = lchunk // 2
  @functools.partial(
      pl.kernel,
      out_type=jax.ShapeDtypeStruct((lpairs, BATCH, 128), jnp.float32),
      mesh=_mesh,
      compiler_params=pltpu.CompilerParams(use_tc_tiling_on_sc=False),
      scratch_types=[
          pltpu.VMEM((SEQ_PER_W, lchunk), jnp.int32),
          pltpu.VMEM((lchunk, EMBED), jnp.float32),
          pltpu.VMEM((lchunk, EMBED), jnp.float32),
          pltpu.VMEM((lchunk, EMBED), jnp.float32),
          pltpu.VMEM((lpairs, 128), jnp.float32),
          pltpu.VMEM((lpairs, 128), jnp.float32),
          pltpu.SemaphoreType.DMA,
          pltpu.SemaphoreType.DMA,
          pltpu.SemaphoreType.DMA,
          pltpu.SemaphoreType.DMA,
      ],
  )
  def _sc_embed_chunk(idx_hbm, tok_hbm, pos_hbm, out_hbm,
                      idx_all, pos_v, rows0, rows1, stg0, stg1,
                      gsem0, gsem1, wsem0, wsem1):
    wid = lax.axis_index("s") * NC + lax.axis_index("c")
    seq_base = wid * SEQ_PER_W
    rows = (rows0, rows1)
    stg = (stg0, stg1)
    gsem = (gsem0, gsem1)
    wsem = (wsem0, wsem1)

    pltpu.sync_copy(
        idx_hbm.at[pl.ds(seq_base, SEQ_PER_W), pl.ds(l0, lchunk)], idx_all)
    pltpu.sync_copy(pos_hbm.at[pl.ds(l0, lchunk)], pos_v)

    def start_gather(s, p):
        pltpu.async_copy(tok_hbm.at[idx_all.at[s]], rows[p], gsem[p])

    def drain_gather(p):
        pltpu.make_async_copy(tok_hbm.at[pl.ds(0, lchunk)], rows[p],
                              gsem[p]).wait()

    def add_and_write(s, p):
        @plsc.parallel_loop(0, lpairs, unroll=ADD_UNROLL)
        def _add(r2):
            for h in range(2):
                for c4 in range(EMBED // 16):
                    src = pl.ds(c4 * 16, 16)
                    dst = pl.ds(h * EMBED + c4 * 16, 16)
                    stg[p][r2, dst] = (rows[p][r2 * 2 + h, src]
                                       + pos_v[r2 * 2 + h, src])

        pltpu.async_copy(stg[p], out_hbm.at[pl.ds(0, lpairs), seq_base + s],
                         wsem[p])

    def drain_write(p):
        pltpu.make_async_copy(stg[p], out_hbm.at[pl.ds(0, lpairs), 0],
                              wsem[p]).wait()

    start_gather(0, 0)

    @pl.loop(0, SEQ_PER_W // 2)
    def _pipeline(h2):
        for par in (0, 1):
            s = h2 * 2 + par
            q = 1 - par

            @pl.when(s + 1 < SEQ_PER_W)
            def _():
                start_gather(s + 1, q)
            drain_gather(par)

            @pl.when(s >= 2)
            def _():
                drain_write(par)
            add_and_write(s, par)

    drain_write(0)
    drain_write(1)

  return _sc_embed_chunk


_sc_chunk_kernels = [_make_sc_embed_chunk(l0, lc) for l0, lc in CHUNKS]


def _tc_relayout_body(in_ref, _prev_ref, out_ref):
    t = in_ref[0]
    tt = jnp.transpose(t)
    out_ref[...] = tt.reshape(2, EMBED, BATCH)


def _tc_relayout_chunk(dense, out_buf, pair_off):
    lpairs = dense.shape[0]
    return pl.pallas_call(
        _tc_relayout_body,
        grid=(lpairs,),
        in_specs=[
            pl.BlockSpec((1, BATCH, 128), lambda i: (i, 0, 0)),
            pl.BlockSpec(memory_space=pl.ANY),
        ],
        out_specs=pl.BlockSpec((2, EMBED, BATCH),
                               lambda i, o=pair_off: (o + i, 0, 0)),
        out_shape=jax.ShapeDtypeStruct((MAXLEN, EMBED, BATCH), jnp.float32),
        input_output_aliases={1: 0},
    )(dense, out_buf)


def _tc_relayout_first_body(in_ref, out_ref):
    t = in_ref[0]
    tt = jnp.transpose(t)
    out_ref[...] = tt.reshape(2, EMBED, BATCH)


def _tc_relayout_first(dense):
    return pl.pallas_call(
        _tc_relayout_first_body,
        grid=(dense.shape[0],),
        in_specs=[pl.BlockSpec((1, BATCH, 128), lambda i: (i, 0, 0))],
        out_specs=pl.BlockSpec((2, EMBED, BATCH), lambda i: (i, 0, 0)),
        out_shape=jax.ShapeDtypeStruct((MAXLEN, EMBED, BATCH), jnp.float32),
    )(dense)


def kernel(x, token_table, pos_table):
    xi = x.astype(jnp.int32)
    chunks = [sc(xi, token_table, pos_table) for sc in _sc_chunk_kernels]
    out_t = _tc_relayout_first(chunks[0])
    pair_off = CHUNKS[0][1] // 2
    for k in range(1, NSPLIT):
        out_t = _tc_relayout_chunk(chunks[k], out_t, pair_off)
        pair_off += CHUNKS[k][1] // 2
    return out_t.transpose(2, 0, 1)


# speedup vs baseline: 1.1444x; 1.0762x over previous
import functools

import jax
import jax.numpy as jnp
from jax import lax
from jax.experimental import pallas as pl
from jax.experimental.pallas import tpu as pltpu
from jax.experimental.pallas import tpu_sc as plsc

MAXLEN = 200
VOCAB = 100000
EMBED = 64
BATCH = 4096

CHUNKS = [(0, 104), (104, 96)]
NSPLIT = len(CHUNKS)

NC, NS = 2, 16
NW = NC * NS
SEQ_PER_W = BATCH // NW
ADD_UNROLL = 4

_mesh = plsc.VectorSubcoreMesh(core_axis_name="c", subcore_axis_name="s")


def _make_sc_embed_chunk(l0, lchunk):
  lpairs = lchunk // 2
  @functools.partial(
      pl.kernel,
      out_type=jax.ShapeDtypeStruct((lpairs, BATCH, 128), jnp.float32),
      mesh=_mesh,
      compiler_params=pltpu.CompilerParams(use_tc_tiling_on_sc=False),
      scratch_types=[
          pltpu.VMEM((SEQ_PER_W, lchunk), jnp.int32),
          pltpu.VMEM((lchunk, EMBED), jnp.float32),
          pltpu.VMEM((lchunk, EMBED), jnp.float32),
          pltpu.VMEM((lchunk, EMBED), jnp.float32),
          pltpu.VMEM((lpairs, 128), jnp.float32),
          pltpu.VMEM((lpairs, 128), jnp.float32),
          pltpu.SemaphoreType.DMA,
          pltpu.SemaphoreType.DMA,
          pltpu.SemaphoreType.DMA,
          pltpu.SemaphoreType.DMA,
      ],
  )
  def _sc_embed_chunk(idx_hbm, tok_hbm, pos_hbm, out_hbm,
                      idx_all, pos_v, rows0, rows1, stg0, stg1,
                      gsem0, gsem1, wsem0, wsem1):
    wid = lax.axis_index("s") * NC + lax.axis_index("c")
    seq_base = wid * SEQ_PER_W
    rows = (rows0, rows1)
    stg = (stg0, stg1)
    gsem = (gsem0, gsem1)
    wsem = (wsem0, wsem1)

    pltpu.sync_copy(
        idx_hbm.at[pl.ds(seq_base, SEQ_PER_W), pl.ds(l0, lchunk)], idx_all)
    pltpu.sync_copy(pos_hbm.at[pl.ds(l0, lchunk)], pos_v)

    def start_gather(s, p):
        pltpu.async_copy(tok_hbm.at[idx_all.at[s]], rows[p], gsem[p])

    def drain_gather(p):
        pltpu.make_async_copy(tok_hbm.at[pl.ds(0, lchunk)], rows[p],
                              gsem[p]).wait()

    def add_and_write(s, p):
        @plsc.parallel_loop(0, lpairs, unroll=ADD_UNROLL)
        def _add(r2):
            for h in range(2):
                for c4 in range(EMBED // 16):
                    src = pl.ds(c4 * 16, 16)
                    dst = pl.ds(h * EMBED + c4 * 16, 16)
                    stg[p][r2, dst] = (rows[p][r2 * 2 + h, src]
                                       + pos_v[r2 * 2 + h, src])

        pltpu.async_copy(stg[p], out_hbm.at[pl.ds(0, lpairs), seq_base + s],
                         wsem[p])

    def drain_write(p):
        pltpu.make_async_copy(stg[p], out_hbm.at[pl.ds(0, lpairs), 0],
                              wsem[p]).wait()

    start_gather(0, 0)

    @pl.loop(0, SEQ_PER_W // 2)
    def _pipeline(h2):
        for par in (0, 1):
            s = h2 * 2 + par
            q = 1 - par

            @pl.when(s + 1 < SEQ_PER_W)
            def _():
                start_gather(s + 1, q)
            drain_gather(par)

            @pl.when(s >= 2)
            def _():
                drain_write(par)
            add_and_write(s, par)

    drain_write(0)
    drain_write(1)

  return _sc_embed_chunk


_sc_chunk_kernels = [_make_sc_embed_chunk(l0, lc) for l0, lc in CHUNKS]


def _tc_relayout_body(in_ref, _prev_ref, out_ref):
    t = in_ref[0]
    tt = jnp.transpose(t)
    out_ref[...] = tt.reshape(2, EMBED, BATCH)


def _tc_relayout_chunk(dense, out_buf, pair_off):
    lpairs = dense.shape[0]
    return pl.pallas_call(
        _tc_relayout_body,
        grid=(lpairs,),
        in_specs=[
            pl.BlockSpec((1, BATCH, 128), lambda i: (i, 0, 0)),
            pl.BlockSpec(memory_space=pl.ANY),
        ],
        out_specs=pl.BlockSpec((2, EMBED, BATCH),
                               lambda i, o=pair_off: (o + i, 0, 0)),
        out_shape=jax.ShapeDtypeStruct((MAXLEN, EMBED, BATCH), jnp.float32),
        input_output_aliases={1: 0},
    )(dense, out_buf)


def _tc_relayout_first_body(in_ref, out_ref):
    t = in_ref[0]
    tt = jnp.transpose(t)
    out_ref[...] = tt.reshape(2, EMBED, BATCH)


def _tc_relayout_first(dense):
    return pl.pallas_call(
        _tc_relayout_first_body,
        grid=(dense.shape[0],),
        in_specs=[pl.BlockSpec((1, BATCH, 128), lambda i: (i, 0, 0))],
        out_specs=pl.BlockSpec((2, EMBED, BATCH), lambda i: (i, 0, 0)),
        out_shape=jax.ShapeDtypeStruct((MAXLEN, EMBED, BATCH), jnp.float32),
    )(dense)


def kernel(x, token_table, pos_table):
    xi = x.astype(jnp.int32)
    chunks = [sc(xi, token_table, pos_table) for sc in _sc_chunk_kernels]
    out_t = _tc_relayout_first(chunks[0])
    pair_off = CHUNKS[0][1] // 2
    for k in range(1, NSPLIT):
        out_t = _tc_relayout_chunk(chunks[k], out_t, pair_off)
        pair_off += CHUNKS[k][1] // 2
    return out_t.transpose(2, 0, 1)
